# baseline (speedup 1.0000x reference)
import functools

import jax
import jax.numpy as jnp
from jax import lax
from jax.experimental import pallas as pl
from jax.experimental.pallas import tpu as pltpu
from jax.experimental.pallas import tpu_sc as plsc

BETA = 0.25
_B, _Q, _K, _N = 8, 64, 512, 2049
_C, _T = 256, 2048
_CHUNK = 256
_GRID = _T // _CHUNK

_NW = 32
_COLS = 128
_BH = 4
_LG = _COLS // 16
_UNROLL = 2


def _tc_body(qp_ref, tgt_ref, out_ref):
    i = pl.program_id(0)

    qp = qp_ref[...]
    m = jnp.max(qp, axis=1, keepdims=True)
    s = jnp.sum(jnp.exp(qp - m), axis=1)
    lse = jnp.log(s) + m[:, 0, :]

    tgt = tgt_ref[:, 0, :]
    cidx = jax.lax.broadcasted_iota(jnp.int32, (_B, _C, _CHUNK), 1)
    picked = jnp.sum(jnp.where(cidx == tgt[:, None, :], qp, 0.0), axis=1)

    contrib = jnp.sum(picked - lse)

    @pl.when(i == 0)
    def _():
        out_ref[0, 0] = 0.0

    out_ref[0, 0] += contrib


def _sc_body(ze_hbm, emb_hbm, out_hbm, ze_v, emb_v, acc_v):
    s = lax.axis_index("s")
    c = lax.axis_index("c")
    wid = s * 2 + c
    cbase = pl.multiple_of(s * _COLS, 128)
    bbase = c * _BH
    pltpu.sync_copy(ze_hbm.at[pl.ds(bbase, _BH), :, pl.ds(cbase, _COLS)], ze_v)
    pltpu.sync_copy(emb_hbm.at[:, pl.ds(cbase, _COLS)], emb_v)

    qinv = 1.0 / float(_Q)
    acc = jnp.zeros((16,), jnp.float32)
    for g in range(_LG):
        sl = pl.ds(g * 16, 16)

        xs, s2s = [], []
        for b in range(_BH):
            def qbody(q, carry):
                v = ze_v[b, q, sl]
                return carry[0] + v, carry[1] + v * v
            z = jnp.zeros((16,), jnp.float32)
            s1, s2 = lax.fori_loop(0, _Q, qbody, (z, z))
            xs.append(s1 * qinv)
            s2s.append(s2)

        def kbody(i, mns):
            out = list(mns)
            for j in range(_UNROLL):
                k = i * _UNROLL + j
                e = emb_v[k, sl]
                for b in range(_BH):
                    out[b] = jnp.minimum(out[b], jnp.abs(e - xs[b]))
            return tuple(out)

        big = jnp.full((16,), 3e38, jnp.float32)
        mns = lax.fori_loop(0, _K // _UNROLL, kbody, (big,) * _BH)

        for b in range(_BH):
            x = xs[b]
            acc = acc + (float(_Q) * (mns[b] * mns[b] - x * x) + s2s[b])

    acc_v[...] = acc
    pltpu.sync_copy(acc_v, out_hbm.at[wid])


@functools.partial(jax.jit, static_argnames=("interpret",))
def kernel(quant_pred, target_wav, ze, emb, interpret=False):
    tgt = target_wav.astype(jnp.int32)

    sc_fn = pl.kernel(
        _sc_body,
        out_type=jax.ShapeDtypeStruct((_NW, 16), jnp.float32),
        mesh=plsc.VectorSubcoreMesh(core_axis_name="c", subcore_axis_name="s"),
        scratch_types=[
            pltpu.VMEM((_BH, _Q, _COLS), jnp.float32),
            pltpu.VMEM((_K, _COLS), jnp.float32),
            pltpu.VMEM((16,), jnp.float32),
        ],
        interpret=interpret,
    )
    minl2_parts = sc_fn(ze, emb)

    tc_sum = pl.pallas_call(
        _tc_body,
        grid=(_GRID,),
        in_specs=[
            pl.BlockSpec((_B, _C, _CHUNK), lambda i: (0, 0, i)),
            pl.BlockSpec((_B, 1, _CHUNK), lambda i: (0, 0, i)),
        ],
        out_specs=pl.BlockSpec(
            (1, 1), lambda i: (0, 0), memory_space=pltpu.SMEM
        ),
        out_shape=jax.ShapeDtypeStruct((1, 1), jnp.float32),
        interpret=interpret,
    )(quant_pred, tgt)

    total = tc_sum[0, 0] + (1.0 + BETA) * jnp.sum(minl2_parts)
    return total / (_B * _T)


# speedup vs baseline: 2.1702x; 2.1702x over previous
import functools

import jax
import jax.numpy as jnp
from jax.experimental import pallas as pl
from jax.experimental.pallas import tpu as pltpu

BETA = 0.25
_B, _Q, _K = 8, 64, 512
_C, _T = 256, 2048
_CHUNK = 512
_GRID = _T // _CHUNK


def _body(qp_ref, tgt_ref, ze_ref, emb_ref, out_ref):
    i = pl.program_id(0)

    qp = qp_ref[...]
    m = jnp.max(qp, axis=1, keepdims=True)
    s = jnp.sum(jnp.exp(qp - m), axis=1)
    lse = jnp.log(s) + m[:, 0, :]

    tgt = tgt_ref[:, 0, :]
    cidx = jax.lax.broadcasted_iota(jnp.int32, (_B, _C, _CHUNK), 1)
    picked = jnp.sum(jnp.where(cidx == tgt[:, None, :], qp, 0.0), axis=1)

    ze = ze_ref[...]
    s1 = jnp.sum(ze, axis=1)
    s2 = jnp.sum(ze * ze, axis=1)

    emb = emb_ref[...]
    a = _Q * emb * emb
    e2 = 2.0 * emb
    mins = []
    for b in range(_B):
        d = a - e2 * s1[b][None, :]
        mins.append(jnp.min(d, axis=0))
    minl2 = jnp.stack(mins, axis=0) + s2

    contrib = jnp.sum(picked - lse + (1.0 + BETA) * minl2)

    @pl.when(i == 0)
    def _():
        out_ref[0, 0] = 0.0

    out_ref[0, 0] += contrib


@functools.partial(jax.jit, static_argnames=("interpret",))
def kernel(quant_pred, target_wav, ze, emb, interpret=False):
    tgt = target_wav.astype(jnp.int32)
    total = pl.pallas_call(
        _body,
        grid=(_GRID,),
        in_specs=[
            pl.BlockSpec((_B, _C, _CHUNK), lambda i: (0, 0, i)),
            pl.BlockSpec((_B, 1, _CHUNK), lambda i: (0, 0, i)),
            pl.BlockSpec((_B, _Q, _CHUNK), lambda i: (0, 0, i)),
            pl.BlockSpec((_K, _CHUNK), lambda i: (0, i)),
        ],
        out_specs=pl.BlockSpec(
            (1, 1), lambda i: (0, 0), memory_space=pltpu.SMEM
        ),
        out_shape=jax.ShapeDtypeStruct((1, 1), jnp.float32),
        interpret=interpret,
    )(quant_pred, tgt, ze, emb)
    return total[0, 0] / (_B * _T)
